# baseline (speedup 1.0000x reference)
import jax
import jax.numpy as jnp
from jax.experimental import pallas as pl
from jax.experimental.pallas import tpu as pltpu

_ROWS = 100000
_BLK = 4000


def _copy_kernel(u_in, i_in, u_out, i_out):
    u_out[...] = u_in[...]
    i_out[...] = i_in[...]


def kernel(user_weight, item_weight):
    grid = _ROWS // _BLK
    spec = pl.BlockSpec((_BLK, 64), lambda n: (n, 0))
    return pl.pallas_call(
        _copy_kernel,
        grid=(grid,),
        out_shape=(
            jax.ShapeDtypeStruct(user_weight.shape, user_weight.dtype),
            jax.ShapeDtypeStruct(item_weight.shape, item_weight.dtype),
        ),
        in_specs=[spec, spec],
        out_specs=(spec, spec),
    )(user_weight, item_weight)
